# baseline (speedup 1.0000x reference)
import functools

import jax
import jax.numpy as jnp
from jax import lax
from jax.experimental import pallas as pl
from jax.experimental.pallas import tpu as pltpu
from jax.experimental.pallas import tpu_sc as plsc

BETA = 0.25
K = 1024
D = 64
N = 16384

NH = N // 2
BN = 4096

NC = 2
NS = 16
NW = NC * NS
GCH = 128


def _k1_body(zb_ref, cb_ref, idx_ref, loss_ref, b2_ref):
    i = pl.program_id(0)
    a = zb_ref[...]
    cb = cb_ref[...]
    a2 = jnp.sum(a * a, axis=1, keepdims=True)

    @pl.when(i == 0)
    def _b2():
        b2_ref[...] = jnp.sum(cb * cb, axis=1)[None, :]

    b2 = b2_ref[...]
    s = lax.dot_general(a, cb, (((1,), (1,)), ((), ())),
                        preferred_element_type=jnp.float32)
    d2 = a2 + b2 - 2.0 * s
    dist = jnp.sqrt(jnp.maximum(d2, 0.0))
    dmin = jnp.min(dist, axis=1, keepdims=True)
    lanes = lax.broadcasted_iota(jnp.int32, (BN, K), 1)
    cand = jnp.where(dist == dmin, lanes, K)
    idx_ref[...] = jnp.min(cand, axis=1, keepdims=True)
    part = jnp.sum(dmin * dmin, axis=0, keepdims=True)

    @pl.when(i == 0)
    def _init():
        loss_ref[...] = part

    @pl.when(i > 0)
    def _acc():
        loss_ref[...] += part


def _k1_call(z_flat, codebook, half, interpret=False):
    grid = NH // BN
    base = half * (NH // BN)
    return pl.pallas_call(
        _k1_body,
        interpret=interpret,
        grid=(grid,),
        in_specs=[
            pl.BlockSpec((BN, D), lambda i: (i + base, 0)),
            pl.BlockSpec((K, D), lambda i: (0, 0)),
        ],
        out_specs=[
            pl.BlockSpec((BN, 1), lambda i: (i, 0)),
            pl.BlockSpec((1, 1), lambda i: (0, 0)),
        ],
        out_shape=[
            jax.ShapeDtypeStruct((NH, 1), jnp.int32),
            jax.ShapeDtypeStruct((1, 1), jnp.float32),
        ],
        scratch_shapes=[pltpu.VMEM((1, K), jnp.float32)],
    )(z_flat, codebook)


BPW = NH // NW
CPW = BPW // GCH


@functools.cache
def _k2_build():
    @functools.partial(
        pl.kernel,
        mesh=plsc.VectorSubcoreMesh(core_axis_name="c", subcore_axis_name="s"),
        compiler_params=pltpu.CompilerParams(use_tc_tiling_on_sc=False,
                                             needs_layout_passes=False),
        out_type=[
            jax.ShapeDtypeStruct((NH, D), jnp.float32),
            jax.ShapeDtypeStruct((NW, K), jnp.int32),
        ],
        scratch_types=[
            pltpu.VMEM((CPW, GCH), jnp.int32),
            pltpu.VMEM((BPW, D), jnp.float32),
            pltpu.VMEM((K,), jnp.int32),
            pltpu.SemaphoreType.DMA,
        ],
    )
    def _k2(cb_hbm, idx_hbm, out_hbm, hist_hbm, idx_v, rows_v, hist_v, sem):
        wid = lax.axis_index("s") * NC + lax.axis_index("c")
        pltpu.sync_copy(idx_hbm.at[pl.ds(wid * CPW, CPW)], idx_v)
        copies = [
            pltpu.async_copy(cb_hbm.at[idx_v.at[j]],
                             rows_v.at[pl.ds(j * GCH, GCH)], sem)
            for j in range(CPW)
        ]
        zeros = jnp.zeros((16,), jnp.int32)
        ones = jnp.ones((16,), jnp.int32)
        for t in range(K // 16):
            hist_v[pl.ds(t * 16, 16)] = zeros
        for j in range(CPW):
            for t in range(GCH // 16):
                v = idx_v[j, pl.ds(t * 16, 16)]
                plsc.addupdate_scatter(hist_v, [v], ones)
        pltpu.sync_copy(hist_v, hist_hbm.at[wid])
        for c in copies:
            c.wait()
        pltpu.sync_copy(rows_v, out_hbm.at[pl.ds(wid * BPW, BPW)])

    return _k2


def kernel(z, codebook):
    z_t = jnp.moveaxis(z, 1, -1)
    z_shape = z_t.shape
    z_flat = z_t.reshape(-1, D)

    k2 = _k2_build()
    idx_a, loss_a = _k1_call(z_flat, codebook, 0)
    zq_a, hist_a = k2(codebook, idx_a.reshape(NH // GCH, GCH))
    idx_b, loss_b = _k1_call(z_flat, codebook, 1)
    zq_b, hist_b = k2(codebook, idx_b.reshape(NH // GCH, GCH))

    nearest_embs = jnp.concatenate(
        [idx_a.reshape(-1), idx_b.reshape(-1)])
    counts = jnp.sum(hist_a, axis=0) + jnp.sum(hist_b, axis=0)

    m = (loss_a[0, 0] + loss_b[0, 0]) / jnp.float32(N * D)
    loss = m + BETA * m

    e_mean = counts.astype(jnp.float32) / nearest_embs.size
    perplexity = jnp.exp(-jnp.sum(e_mean * jnp.log(e_mean + 1e-10)))

    z_q = jnp.concatenate([zq_a, zq_b], axis=0)
    z_q_out = jnp.moveaxis(z_q.reshape(z_shape), -1, 1)
    return (z_q_out, loss, perplexity, nearest_embs, z_flat)


# speedup vs baseline: 1.1159x; 1.1159x over previous
import functools

import jax
import jax.numpy as jnp
from jax import lax
from jax.experimental import pallas as pl
from jax.experimental.pallas import tpu as pltpu
from jax.experimental.pallas import tpu_sc as plsc

BETA = 0.25
K = 1024
D = 64
N = 16384

BN = 4096

NC = 2
NS = 16
NW = NC * NS
GCH = 128


def _k1_body(zb_ref, cb_ref, idx_ref, loss_ref, b2_ref):
    i = pl.program_id(0)
    a = zb_ref[...]
    cb = cb_ref[...]
    a2 = jnp.sum(a * a, axis=1, keepdims=True)

    @pl.when(i == 0)
    def _b2():
        b2_ref[...] = jnp.sum(cb * cb, axis=1)[None, :]

    b2 = b2_ref[...]
    s = lax.dot_general(a, cb, (((1,), (1,)), ((), ())),
                        preferred_element_type=jnp.float32)
    d2 = a2 + b2 - 2.0 * s
    dist = jnp.sqrt(jnp.maximum(d2, 0.0))
    dmin = jnp.min(dist, axis=1, keepdims=True)
    lanes = lax.broadcasted_iota(jnp.int32, (BN, K), 1)
    cand = jnp.where(dist == dmin, lanes, K)
    idx_ref[...] = jnp.min(cand, axis=1, keepdims=True)
    part = jnp.sum(dmin * dmin, axis=0, keepdims=True)

    @pl.when(i == 0)
    def _init():
        loss_ref[...] = part

    @pl.when(i > 0)
    def _acc():
        loss_ref[...] += part


def _k1_call(z_flat, codebook, interpret=False):
    grid = N // BN
    base = 0
    return pl.pallas_call(
        _k1_body,
        interpret=interpret,
        grid=(grid,),
        in_specs=[
            pl.BlockSpec((BN, D), lambda i: (i + base, 0)),
            pl.BlockSpec((K, D), lambda i: (0, 0)),
        ],
        out_specs=[
            pl.BlockSpec((BN, 1), lambda i: (i, 0)),
            pl.BlockSpec((1, 1), lambda i: (0, 0)),
        ],
        out_shape=[
            jax.ShapeDtypeStruct((N, 1), jnp.int32),
            jax.ShapeDtypeStruct((1, 1), jnp.float32),
        ],
        scratch_shapes=[pltpu.VMEM((1, K), jnp.float32)],
    )(z_flat, codebook)


BPW = N // NW
CPW = BPW // GCH


@functools.cache
def _k2_build():
    @functools.partial(
        pl.kernel,
        mesh=plsc.VectorSubcoreMesh(core_axis_name="c", subcore_axis_name="s"),
        compiler_params=pltpu.CompilerParams(use_tc_tiling_on_sc=False,
                                             needs_layout_passes=False),
        out_type=[
            jax.ShapeDtypeStruct((N, D), jnp.float32),
            jax.ShapeDtypeStruct((NW, K), jnp.int32),
        ],
        scratch_types=[
            pltpu.VMEM((CPW, GCH), jnp.int32),
            pltpu.VMEM((BPW, D), jnp.float32),
            pltpu.VMEM((K,), jnp.int32),
            pltpu.SemaphoreType.DMA,
        ],
    )
    def _k2(cb_hbm, idx_hbm, out_hbm, hist_hbm, idx_v, rows_v, hist_v, sem):
        wid = lax.axis_index("s") * NC + lax.axis_index("c")
        pltpu.sync_copy(idx_hbm.at[pl.ds(wid * CPW, CPW)], idx_v)
        copies = [
            pltpu.async_copy(cb_hbm.at[idx_v.at[j]],
                             rows_v.at[pl.ds(j * GCH, GCH)], sem)
            for j in range(CPW)
        ]
        zeros = jnp.zeros((16,), jnp.int32)
        ones = jnp.ones((16,), jnp.int32)
        for t in range(K // 16):
            hist_v[pl.ds(t * 16, 16)] = zeros
        for j in range(CPW):
            for t in range(GCH // 16):
                v = idx_v[j, pl.ds(t * 16, 16)]
                plsc.addupdate_scatter(hist_v, [v], ones)
        pltpu.sync_copy(hist_v, hist_hbm.at[wid])
        for c in copies:
            c.wait()
        pltpu.sync_copy(rows_v, out_hbm.at[pl.ds(wid * BPW, BPW)])

    return _k2


def kernel(z, codebook):
    z_t = jnp.moveaxis(z, 1, -1)
    z_shape = z_t.shape
    z_flat = z_t.reshape(-1, D)

    idx2d, loss_sum = _k1_call(z_flat, codebook)
    nearest_embs = idx2d.reshape(-1)

    z_q, hists = _k2_build()(codebook, nearest_embs.reshape(N // GCH, GCH))
    counts = jnp.sum(hists, axis=0)

    m = loss_sum[0, 0] / jnp.float32(N * D)
    loss = m + BETA * m

    e_mean = counts.astype(jnp.float32) / nearest_embs.size
    perplexity = jnp.exp(-jnp.sum(e_mean * jnp.log(e_mean + 1e-10)))

    z_q_out = jnp.moveaxis(z_q.reshape(z_shape), -1, 1)
    return (z_q_out, loss, perplexity, nearest_embs, z_flat)


# speedup vs baseline: 1.1395x; 1.0211x over previous
import functools

import jax
import jax.numpy as jnp
from jax import lax
from jax.experimental import pallas as pl
from jax.experimental.pallas import tpu as pltpu
from jax.experimental.pallas import tpu_sc as plsc

BETA = 0.25
K = 1024
D = 64
N = 16384

BN = 4096

NC = 2
NS = 16
NW = NC * NS
GCH = 128


def _k1_body(zb_ref, cb_ref, idx_ref, loss_ref, b2_ref, cb2_ref):
    i = pl.program_id(0)
    a = zb_ref[...]
    cb = cb_ref[...]
    a2 = jnp.sum(a * a, axis=1, keepdims=True)

    @pl.when(i == 0)
    def _b2():
        b2_ref[...] = jnp.sum(cb * cb, axis=1)[None, :]
        cb2_ref[...] = cb + cb

    b2 = b2_ref[...]
    s2 = lax.dot_general(a, cb2_ref[...], (((1,), (1,)), ((), ())),
                         preferred_element_type=jnp.float32)
    d2 = a2 + b2 - s2
    dist = jnp.sqrt(jnp.maximum(d2, 0.0))
    dmin = jnp.min(dist, axis=1, keepdims=True)
    lanes = lax.broadcasted_iota(jnp.int32, (BN, K), 1)
    cand = jnp.where(dist == dmin, lanes, K)
    idx_ref[...] = jnp.min(cand, axis=1, keepdims=True)
    part = jnp.sum(dmin * dmin, axis=0, keepdims=True)

    @pl.when(i == 0)
    def _init():
        loss_ref[...] = part

    @pl.when(i > 0)
    def _acc():
        loss_ref[...] += part


def _k1_call(z_flat, codebook, interpret=False):
    grid = N // BN
    base = 0
    return pl.pallas_call(
        _k1_body,
        interpret=interpret,
        grid=(grid,),
        in_specs=[
            pl.BlockSpec((BN, D), lambda i: (i + base, 0)),
            pl.BlockSpec((K, D), lambda i: (0, 0)),
        ],
        out_specs=[
            pl.BlockSpec((BN, 1), lambda i: (i, 0)),
            pl.BlockSpec((1, 1), lambda i: (0, 0)),
        ],
        out_shape=[
            jax.ShapeDtypeStruct((N, 1), jnp.int32),
            jax.ShapeDtypeStruct((1, 1), jnp.float32),
        ],
        scratch_shapes=[pltpu.VMEM((1, K), jnp.float32),
                        pltpu.VMEM((K, D), jnp.float32)],
    )(z_flat, codebook)


BPW = N // NW
CPW = BPW // GCH


@functools.cache
def _k2_build():
    @functools.partial(
        pl.kernel,
        mesh=plsc.VectorSubcoreMesh(core_axis_name="c", subcore_axis_name="s"),
        compiler_params=pltpu.CompilerParams(use_tc_tiling_on_sc=False,
                                             needs_layout_passes=False),
        out_type=[
            jax.ShapeDtypeStruct((N, D), jnp.float32),
            jax.ShapeDtypeStruct((NW, K), jnp.int32),
        ],
        scratch_types=[
            pltpu.VMEM((CPW, GCH), jnp.int32),
            pltpu.VMEM((BPW, D), jnp.float32),
            pltpu.VMEM((K,), jnp.int32),
            pltpu.SemaphoreType.DMA,
        ],
    )
    def _k2(cb_hbm, idx_hbm, out_hbm, hist_hbm, idx_v, rows_v, hist_v, sem):
        wid = lax.axis_index("s") * NC + lax.axis_index("c")
        pltpu.sync_copy(idx_hbm.at[pl.ds(wid * CPW, CPW)], idx_v)
        copies = [
            pltpu.async_copy(cb_hbm.at[idx_v.at[j]],
                             rows_v.at[pl.ds(j * GCH, GCH)], sem)
            for j in range(CPW)
        ]
        zeros = jnp.zeros((16,), jnp.int32)
        ones = jnp.ones((16,), jnp.int32)
        for t in range(K // 16):
            hist_v[pl.ds(t * 16, 16)] = zeros
        for j in range(CPW):
            for t in range(GCH // 16):
                v = idx_v[j, pl.ds(t * 16, 16)]
                plsc.addupdate_scatter(hist_v, [v], ones)
        pltpu.sync_copy(hist_v, hist_hbm.at[wid])
        for c in copies:
            c.wait()
        pltpu.sync_copy(rows_v, out_hbm.at[pl.ds(wid * BPW, BPW)])

    return _k2


def kernel(z, codebook):
    z_t = jnp.moveaxis(z, 1, -1)
    z_shape = z_t.shape
    z_flat = z_t.reshape(-1, D)

    idx2d, loss_sum = _k1_call(z_flat, codebook)
    nearest_embs = idx2d.reshape(-1)

    z_q, hists = _k2_build()(codebook, nearest_embs.reshape(N // GCH, GCH))
    counts = jnp.sum(hists, axis=0)

    m = loss_sum[0, 0] / jnp.float32(N * D)
    loss = m + BETA * m

    e_mean = counts.astype(jnp.float32) / nearest_embs.size
    perplexity = jnp.exp(-jnp.sum(e_mean * jnp.log(e_mean + 1e-10)))

    z_q_out = jnp.moveaxis(z_q.reshape(z_shape), -1, 1)
    return (z_q_out, loss, perplexity, nearest_embs, z_flat)
